# baseline (speedup 1.0000x reference)
import functools

import jax
import jax.numpy as jnp
from jax import lax
from jax.experimental import pallas as pl
from jax.experimental.pallas import tpu as pltpu
from jax.experimental.pallas import tpu_sc as plsc

B, H, W, K, J = 8, 256, 256, 100, 17
HW = H * W
NMAPS = B + B * J
NV = HW // 16
CAP = 8192
CAPP = CAP + 16


def _sig_nms_body(x_ref, s_ref, sc_ref):
    x = x_ref[0]
    s = jax.nn.sigmoid(x)
    s_ref[0] = s
    neg = jnp.full((H, 1), -jnp.inf, x.dtype)
    mrow = jnp.maximum(s, jnp.concatenate([s[:, 1:], neg], axis=1))
    mrow = jnp.maximum(mrow, jnp.concatenate([neg, s[:, :-1]], axis=1))
    negr = jnp.full((1, W), -jnp.inf, x.dtype)
    mcol = jnp.maximum(mrow, jnp.concatenate([mrow[1:, :], negr], axis=0))
    mcol = jnp.maximum(mcol, jnp.concatenate([negr, mrow[:-1, :]], axis=0))
    keep = (mcol == s)
    sc_ref[0] = jnp.where(keep, s, 0.0)


def _sig_nms(x):
    b, c = x.shape[0], x.shape[1]
    xf = x.reshape(b * c, H, W)
    out = pl.pallas_call(
        _sig_nms_body,
        grid=(b * c,),
        in_specs=[pl.BlockSpec((1, H, W), lambda i: (i, 0, 0))],
        out_specs=[pl.BlockSpec((1, H, W), lambda i: (i, 0, 0))] * 2,
        out_shape=[jax.ShapeDtypeStruct((b * c, H, W), x.dtype)] * 2,
    )(xf)
    return out[0].reshape(x.shape), out[1].reshape(b * c, HW)


def _rung(k):
    t = jnp.where(k == 0, jnp.float32(0.999), jnp.float32(1e-8))
    t = jnp.where(k == 1, jnp.float32(0.99), t)
    t = jnp.where(k == 2, jnp.float32(0.9), t)
    return t


def _sc_topk(hm_sc, hp_sc):
    mesh = plsc.VectorSubcoreMesh(core_axis_name="c", subcore_axis_name="s")

    @functools.partial(
        pl.kernel,
        out_type=[
            jax.ShapeDtypeStruct((NMAPS, 128), jnp.float32),
            jax.ShapeDtypeStruct((NMAPS, 128), jnp.int32),
        ],
        mesh=mesh,
        compiler_params=pltpu.CompilerParams(needs_layout_passes=False),
        scratch_types=[
            pltpu.VMEM((HW,), jnp.float32),
            pltpu.VMEM((CAPP,), jnp.float32),
            pltpu.VMEM((CAPP,), jnp.int32),
            pltpu.VMEM((128,), jnp.float32),
            pltpu.VMEM((128,), jnp.int32),
            pltpu.VMEM((128,), jnp.float32),
            pltpu.VMEM((128,), jnp.int32),
            pltpu.VMEM((112,), jnp.int32),
            pltpu.VMEM((128,), jnp.float32),
            pltpu.VMEM((128,), jnp.int32),
        ],
    )
    def topk_kernel(hm_hbm, hp_hbm, outv_hbm, outi_hbm,
                    map_v, cand_v, cidx_v, tie_v, tidx_v,
                    fin_v, fidx_v, rank_v, outv_v, outi_v):
        w = lax.axis_index("s") * 2 + lax.axis_index("c")
        laneiota = lax.iota(jnp.int32, 16)

        def compact_pass(t, nvec):
            def body(i, cnt):
                v = map_v[pl.ds(i * 16, 16)]
                msk = v >= t
                n = plsc.all_reduce_population_count(msk)[0]

                @pl.when(n > 0)
                def _():
                    off = jnp.minimum(cnt, CAP)
                    plsc.store_compressed(cand_v.at[pl.ds(off, 16)], v,
                                          mask=msk)
                    idxv = laneiota + i * 16
                    plsc.store_compressed(cidx_v.at[pl.ds(off, 16)], idxv,
                                          mask=msk)
                return cnt + n
            return lax.fori_loop(0, nvec, body, jnp.int32(0))

        def count_ge(tb, cnt, nv_cand, nv_map):
            def cbody(i, acc):
                v = cand_v[pl.ds(i * 16, 16)]
                bits = plsc.bitcast(v, jnp.int32)
                valid = (i * 16 + laneiota) < cnt
                m = (bits >= tb) & valid
                return acc + plsc.all_reduce_population_count(m)[0]
            acc = lax.fori_loop(0, nv_cand, cbody, jnp.int32(0))

            def mbody(i, acc):
                v = map_v[pl.ds(i * 16, 16)]
                bits = plsc.bitcast(v, jnp.int32)
                m = bits >= tb
                return acc + plsc.all_reduce_population_count(m)[0]
            return lax.fori_loop(0, nv_map, mbody, acc)

        def process(g):
            @pl.when(g < B)
            def _():
                pltpu.sync_copy(hm_hbm.at[jnp.minimum(g, B - 1)], map_v)

            @pl.when(g >= B)
            def _():
                pltpu.sync_copy(hp_hbm.at[jnp.maximum(g - B, 0)], map_v)

            cnt = compact_pass(_rung(jnp.int32(2)), NV)

            def step(_, st):
                k, c = st
                ok = (c >= K) & (c <= CAP)
                k2 = jnp.where(c > CAP, k - 1, jnp.where(c < K, k + 1, k))
                live = (~ok) & (k2 >= 0) & (k2 <= 3)
                c2 = compact_pass(_rung(k2), jnp.where(live, NV, 0))
                return (jnp.where(ok, k, k2), jnp.where(live, c2, c))
            _, cnt = lax.fori_loop(0, 3, step, (jnp.int32(2), cnt))
            fb = ~((cnt >= K) & (cnt <= CAP))

            nv_cand = jnp.where(fb, 0, (cnt + 15) // 16)
            nv_map = jnp.where(fb, NV, 0)

            def bbody(_, st):
                lo, hi = st
                mid = lo + (hi - lo) // 2
                n = count_ge(mid, cnt, nv_cand, nv_map)
                return (jnp.where(n >= K, mid, lo),
                        jnp.where(n >= K, hi, mid))
            vkb, _ = lax.fori_loop(
                0, 31, bbody,
                (jnp.where(fb, jnp.int32(0), jnp.int32(1)),
                 jnp.int32(0x3F800001)))

            def hc_body(i, p):
                v = cand_v[pl.ds(i * 16, 16)]
                ix = cidx_v[pl.ds(i * 16, 16)]
                bits = plsc.bitcast(v, jnp.int32)
                valid = (i * 16 + laneiota) < cnt
                m = (bits > vkb) & valid
                plsc.store_compressed(fin_v.at[pl.ds(p, 16)], v, mask=m)
                plsc.store_compressed(fidx_v.at[pl.ds(p, 16)], ix, mask=m)
                return p + plsc.all_reduce_population_count(m)[0]
            nhi = lax.fori_loop(0, nv_cand, hc_body, jnp.int32(0))

            def hm_body(i, p):
                v = map_v[pl.ds(i * 16, 16)]
                bits = plsc.bitcast(v, jnp.int32)
                m = bits > vkb
                plsc.store_compressed(fin_v.at[pl.ds(p, 16)], v, mask=m)
                plsc.store_compressed(fidx_v.at[pl.ds(p, 16)],
                                      laneiota + i * 16, mask=m)
                return p + plsc.all_reduce_population_count(m)[0]
            nhi = lax.fori_loop(0, nv_map, hm_body, nhi)

            def tc_body(i, p):
                v = cand_v[pl.ds(i * 16, 16)]
                ix = cidx_v[pl.ds(i * 16, 16)]
                bits = plsc.bitcast(v, jnp.int32)
                valid = (i * 16 + laneiota) < cnt
                m = (bits == vkb) & valid
                off = jnp.minimum(p, 100)
                plsc.store_compressed(tie_v.at[pl.ds(off, 16)], v, mask=m)
                plsc.store_compressed(tidx_v.at[pl.ds(off, 16)], ix, mask=m)
                return p + plsc.all_reduce_population_count(m)[0]
            nt = lax.fori_loop(0, nv_cand, tc_body, jnp.int32(0))

            def tm_body(i, p):
                v = map_v[pl.ds(i * 16, 16)]
                bits = plsc.bitcast(v, jnp.int32)
                m = bits == vkb
                off = jnp.minimum(p, 100)
                plsc.store_compressed(tie_v.at[pl.ds(off, 16)], v, mask=m)
                plsc.store_compressed(tidx_v.at[pl.ds(off, 16)],
                                      laneiota + i * 16, mask=m)
                return p + plsc.all_reduce_population_count(m)[0]
            lax.fori_loop(0, nv_map, tm_body, nt)

            for t in range(7):
                p = t * 16 + laneiota
                cur_v = fin_v[pl.ds(t * 16, 16)]
                cur_i = fidx_v[pl.ds(t * 16, 16)]
                src = jnp.maximum(p - nhi, 0)
                tv = plsc.load_gather(tie_v, [src])
                ti = plsc.load_gather(tidx_v, [src])
                in_hi = p < nhi
                in_tie = p < K
                nv = jnp.where(in_hi, cur_v,
                               jnp.where(in_tie, tv, jnp.float32(-1.0)))
                ni = jnp.where(in_hi, cur_i,
                               jnp.where(in_tie, ti, jnp.int32(0x7FFFFFFF)))
                fin_v[pl.ds(t * 16, 16)] = nv
                fidx_v[pl.ds(t * 16, 16)] = ni
                rank_v[pl.ds(t * 16, 16)] = jnp.zeros((16,), jnp.int32)

            def rbody(j, _):
                vj = fin_v[pl.ds(j, 16)][0]
                ij = fidx_v[pl.ds(j, 16)][0]
                for t in range(7):
                    fv = fin_v[pl.ds(t * 16, 16)]
                    fi = fidx_v[pl.ds(t * 16, 16)]
                    before = (vj > fv) | ((vj == fv) & (ij < fi))
                    acc = rank_v[pl.ds(t * 16, 16)]
                    rank_v[pl.ds(t * 16, 16)] = acc + before.astype(jnp.int32)
                return 0
            lax.fori_loop(0, K, rbody, 0)

            for t in range(7):
                r = rank_v[pl.ds(t * 16, 16)]
                m = r < K
                rc = jnp.minimum(r, jnp.int32(127))
                plsc.store_scatter(outv_v, [rc], fin_v[pl.ds(t * 16, 16)],
                                   mask=m)
                plsc.store_scatter(outi_v, [rc], fidx_v[pl.ds(t * 16, 16)],
                                   mask=m)

            pltpu.sync_copy(outv_v, outv_hbm.at[g])
            pltpu.sync_copy(outi_v, outi_hbm.at[g])

        def round_body(r, _):
            process(w + 32 * r)
            return 0
        nrounds = jnp.where(w < NMAPS - 128, 5, 4)
        lax.fori_loop(0, nrounds, round_body, 0)

    return topk_kernel(hm_sc, hp_sc)


KP = 104


def _decode_body(hx_ref, hy_ref, hs_ref, rx_ref, ry_ref,
                 l_ref, t_ref, r_ref, bo_ref, ox_ref, oy_ref):
    hx = hx_ref[0, 0]
    hy = hy_ref[0, 0]
    hs = hs_ref[0, 0]
    rx = rx_ref[0, 0]
    ry = ry_ref[0, 0]
    l = l_ref[0]
    t = t_ref[0]
    r = r_ref[0]
    bo = bo_ref[0]
    dx = rx - hx
    dy = ry - hy
    dist = jnp.sqrt(dx * dx + dy * dy)
    mind = jnp.min(dist, axis=1, keepdims=True)
    lane = lax.broadcasted_iota(jnp.int32, (KP, 128), 1)
    minidx = jnp.min(jnp.where(dist == mind, lane, jnp.int32(1 << 30)),
                     axis=1, keepdims=True)
    sel = lane == minidx
    zero = jnp.zeros((KP, 128), jnp.float32)
    hxsel = jnp.sum(jnp.where(sel, jnp.broadcast_to(hx, (KP, 128)), zero),
                    axis=1, keepdims=True)
    hysel = jnp.sum(jnp.where(sel, jnp.broadcast_to(hy, (KP, 128)), zero),
                    axis=1, keepdims=True)
    hssel = jnp.sum(jnp.where(sel, jnp.broadcast_to(hs, (KP, 128)), zero),
                    axis=1, keepdims=True)
    m2 = ((hxsel < l) | (hxsel > r) | (hysel < t) | (hysel > bo) |
          (hssel < jnp.float32(0.1)) |
          (mind > jnp.maximum(bo - t, r - l) * jnp.float32(0.3)))
    ox_ref[0, 0] = jnp.where(m2, rx, hxsel)
    oy_ref[0, 0] = jnp.where(m2, ry, hysel)


def _decode(hm_xs, hm_ys, hm_score, rx, ry, bboxes):
    bigf = jnp.float32(1e9)
    row = lambda a, pad: jnp.pad(a, ((0, 0), (0, 0), (0, 128 - K)),
                                 constant_values=pad)[:, :, None, :]
    col = lambda a: jnp.pad(a, ((0, 0), (0, 0), (0, KP - K)))[..., None]
    bcol = lambda a: jnp.pad(a, ((0, 0), (0, KP - K)))[..., None]
    hxr = row(hm_xs, bigf)
    hyr = row(hm_ys, bigf)
    hsr = row(hm_score, 0.0)
    rxc = col(rx)
    ryc = col(ry)
    l = bcol(bboxes[:, :, 0])
    t = bcol(bboxes[:, :, 1])
    r = bcol(bboxes[:, :, 2])
    bo = bcol(bboxes[:, :, 3])
    G = B * J
    rowspec = pl.BlockSpec((1, 1, 1, 128), lambda g: (g // J, g % J, 0, 0))
    colspec = pl.BlockSpec((1, 1, KP, 1), lambda g: (g // J, g % J, 0, 0))
    bspec = pl.BlockSpec((1, KP, 1), lambda g: (g // J, 0, 0))
    ox, oy = pl.pallas_call(
        _decode_body,
        grid=(G,),
        in_specs=[rowspec, rowspec, rowspec, colspec, colspec,
                  bspec, bspec, bspec, bspec],
        out_specs=[colspec, colspec],
        out_shape=[jax.ShapeDtypeStruct((B, J, KP, 1), jnp.float32)] * 2,
    )(hxr, hyr, hsr, rxc, ryc, l, t, r, bo)
    return ox[:, :, :K, 0], oy[:, :, :K, 0]


def _gather_feat(feat, ind):
    b, k = ind.shape
    c = feat.shape[2]
    idx = jnp.broadcast_to(ind[:, :, None], (b, k, c))
    return jnp.take_along_axis(feat, idx, axis=1)


def _transpose_gather(feat, ind):
    b, c, h, w = feat.shape
    feat = jnp.transpose(feat, (0, 2, 3, 1)).reshape(b, h * w, c)
    return _gather_feat(feat, ind)


def kernel(hm, wh, hps, reg, hm_hp, hp_offset):
    hm_s, hm_scores = _sig_nms(hm)
    hm_hp_s, hp_scores = _sig_nms(hm_hp)

    vals, idxs = _sc_topk(hm_scores, hp_scores)
    b = B
    scores = vals[:B, :K]
    inds = idxs[:B, :K]
    hm_score = vals[B:, :K].reshape(b, J, K)
    hm_inds = idxs[B:, :K].reshape(b, J, K)

    ys = (inds // W).astype(jnp.float32)
    xs = (inds % W).astype(jnp.float32)
    clses2 = jnp.zeros((b, K, 1), jnp.float32)

    kps = _transpose_gather(hps, inds)
    kps = kps.at[..., 0::2].add(xs[:, :, None])
    kps = kps.at[..., 1::2].add(ys[:, :, None])
    regg = _transpose_gather(reg, inds)
    xs2 = xs[:, :, None] + regg[:, :, 0:1]
    ys2 = ys[:, :, None] + regg[:, :, 1:2]
    whg = _transpose_gather(wh, inds)
    scores2 = scores[:, :, None]
    bboxes = jnp.concatenate([
        xs2 - whg[..., 0:1] / 2, ys2 - whg[..., 1:2] / 2,
        xs2 + whg[..., 0:1] / 2, ys2 + whg[..., 1:2] / 2], axis=2)
    thresh = 0.1
    kps_t = jnp.transpose(kps.reshape(b, K, J, 2), (0, 2, 1, 3))
    rx = kps_t[..., 0]
    ry = kps_t[..., 1]
    hm_ys = (hm_inds // W).astype(jnp.float32)
    hm_xs = (hm_inds % W).astype(jnp.float32)
    hp_off = _transpose_gather(hp_offset, hm_inds.reshape(b, -1)).reshape(b, J, K, 2)
    hm_xs = hm_xs + hp_off[..., 0]
    hm_ys = hm_ys + hp_off[..., 1]
    mask = (hm_score > thresh).astype(jnp.float32)
    hm_score = (1 - mask) * -1 + mask * hm_score
    hm_ys = (1 - mask) * -10000 + mask * hm_ys
    hm_xs = (1 - mask) * -10000 + mask * hm_xs
    ox, oy = _decode(hm_xs, hm_ys, hm_score, rx, ry, bboxes)
    kps_f = jnp.stack([ox, oy], axis=-1)
    kps_f = jnp.transpose(kps_f, (0, 2, 1, 3)).reshape(b, K, J * 2)
    det = jnp.concatenate([bboxes, scores2, kps_f, clses2], axis=2)
    return (hm_s, wh, hps, reg, hm_hp_s, hp_offset, det)


# speedup vs baseline: 1.0238x; 1.0238x over previous
import functools

import jax
import jax.numpy as jnp
from jax import lax
from jax.experimental import pallas as pl
from jax.experimental.pallas import tpu as pltpu
from jax.experimental.pallas import tpu_sc as plsc

B, H, W, K, J = 8, 256, 256, 100, 17
HW = H * W
NMAPS = B + B * J
NV = HW // 16
CAP = 8192
CAPP = CAP + 80


def _sig_nms_body(x_ref, s_ref, sc_ref):
    x = x_ref[0]
    s = jax.nn.sigmoid(x)
    s_ref[0] = s
    neg = jnp.full((H, 1), -jnp.inf, x.dtype)
    mrow = jnp.maximum(s, jnp.concatenate([s[:, 1:], neg], axis=1))
    mrow = jnp.maximum(mrow, jnp.concatenate([neg, s[:, :-1]], axis=1))
    negr = jnp.full((1, W), -jnp.inf, x.dtype)
    mcol = jnp.maximum(mrow, jnp.concatenate([mrow[1:, :], negr], axis=0))
    mcol = jnp.maximum(mcol, jnp.concatenate([negr, mrow[:-1, :]], axis=0))
    keep = (mcol == s)
    sc_ref[0] = jnp.where(keep, s, 0.0)


def _sig_nms(x):
    b, c = x.shape[0], x.shape[1]
    xf = x.reshape(b * c, H, W)
    out = pl.pallas_call(
        _sig_nms_body,
        grid=(b * c,),
        in_specs=[pl.BlockSpec((1, H, W), lambda i: (i, 0, 0))],
        out_specs=[pl.BlockSpec((1, H, W), lambda i: (i, 0, 0))] * 2,
        out_shape=[jax.ShapeDtypeStruct((b * c, H, W), x.dtype)] * 2,
    )(xf)
    return out[0].reshape(x.shape), out[1].reshape(b * c, HW)


def _rung(k):
    t = jnp.where(k == 0, jnp.float32(0.999), jnp.float32(1e-8))
    t = jnp.where(k == 1, jnp.float32(0.99), t)
    t = jnp.where(k == 2, jnp.float32(0.9), t)
    return t


def _sc_topk(hm_sc, hp_sc):
    mesh = plsc.VectorSubcoreMesh(core_axis_name="c", subcore_axis_name="s")

    @functools.partial(
        pl.kernel,
        out_type=[
            jax.ShapeDtypeStruct((NMAPS, 128), jnp.float32),
            jax.ShapeDtypeStruct((NMAPS, 128), jnp.int32),
        ],
        mesh=mesh,
        compiler_params=pltpu.CompilerParams(needs_layout_passes=False),
        scratch_types=[
            pltpu.VMEM((HW,), jnp.float32),
            pltpu.VMEM((CAPP,), jnp.float32),
            pltpu.VMEM((CAPP,), jnp.int32),
            pltpu.VMEM((128,), jnp.float32),
            pltpu.VMEM((128,), jnp.int32),
            pltpu.VMEM((128,), jnp.float32),
            pltpu.VMEM((128,), jnp.int32),
            pltpu.VMEM((112,), jnp.int32),
            pltpu.VMEM((128,), jnp.float32),
            pltpu.VMEM((128,), jnp.int32),
        ],
    )
    def topk_kernel(hm_hbm, hp_hbm, outv_hbm, outi_hbm,
                    map_v, cand_v, cidx_v, tie_v, tidx_v,
                    fin_v, fidx_v, rank_v, outv_v, outi_v):
        w = lax.axis_index("s") * 2 + lax.axis_index("c")
        laneiota = lax.iota(jnp.int32, 16)

        def compact_pass(t, nvec):
            def body(i8, cnt):
                for u in range(8):
                    i = i8 * 8 + u
                    v = map_v[pl.ds(i * 16, 16)]
                    msk = v >= t
                    n = plsc.all_reduce_population_count(msk)[0]

                    @pl.when(n > 0)
                    def _(v=v, msk=msk, cnt=cnt, i=i):
                        off = jnp.minimum(cnt, CAP)
                        plsc.store_compressed(cand_v.at[pl.ds(off, 16)], v,
                                              mask=msk)
                        idxv = laneiota + i * 16
                        plsc.store_compressed(cidx_v.at[pl.ds(off, 16)],
                                              idxv, mask=msk)
                    cnt = cnt + n
                return cnt
            return lax.fori_loop(0, nvec // 8, body, jnp.int32(0))

        def count_ge(tb, cnt, nv_cand, nv_map):
            def cbody(i4, acc):
                for u in range(4):
                    i = i4 * 4 + u
                    v = cand_v[pl.ds(i * 16, 16)]
                    bits = plsc.bitcast(v, jnp.int32)
                    valid = (i * 16 + laneiota) < cnt
                    m = (bits >= tb) & valid
                    acc = acc + plsc.all_reduce_population_count(m)[0]
                return acc
            acc = lax.fori_loop(0, (nv_cand + 3) // 4, cbody, jnp.int32(0))

            def mbody(i4, acc):
                for u in range(4):
                    i = i4 * 4 + u
                    v = map_v[pl.ds(i * 16, 16)]
                    bits = plsc.bitcast(v, jnp.int32)
                    m = bits >= tb
                    acc = acc + plsc.all_reduce_population_count(m)[0]
                return acc
            return lax.fori_loop(0, nv_map // 4, mbody, acc)

        def process(g):
            @pl.when(g < B)
            def _():
                pltpu.sync_copy(hm_hbm.at[jnp.minimum(g, B - 1)], map_v)

            @pl.when(g >= B)
            def _():
                pltpu.sync_copy(hp_hbm.at[jnp.maximum(g - B, 0)], map_v)

            cnt = compact_pass(_rung(jnp.int32(2)), NV)

            def step(_, st):
                k, c = st
                ok = (c >= K) & (c <= CAP)
                k2 = jnp.where(c > CAP, k - 1, jnp.where(c < K, k + 1, k))
                live = (~ok) & (k2 >= 0) & (k2 <= 3)
                c2 = compact_pass(_rung(k2), jnp.where(live, NV, 0))
                return (jnp.where(ok, k, k2), jnp.where(live, c2, c))
            _, cnt = lax.fori_loop(0, 3, step, (jnp.int32(2), cnt))
            fb = ~((cnt >= K) & (cnt <= CAP))

            nv_cand = jnp.where(fb, 0, (cnt + 15) // 16)
            nv_map = jnp.where(fb, NV, 0)

            def bbody(_, st):
                lo, hi = st
                mid = lo + (hi - lo) // 2
                n = count_ge(mid, cnt, nv_cand, nv_map)
                return (jnp.where(n >= K, mid, lo),
                        jnp.where(n >= K, hi, mid))
            vkb, _ = lax.fori_loop(
                0, 31, bbody,
                (jnp.where(fb, jnp.int32(0), jnp.int32(1)),
                 jnp.int32(0x3F800001)))

            def hc_body(i, p):
                v = cand_v[pl.ds(i * 16, 16)]
                ix = cidx_v[pl.ds(i * 16, 16)]
                bits = plsc.bitcast(v, jnp.int32)
                valid = (i * 16 + laneiota) < cnt
                m = (bits > vkb) & valid
                plsc.store_compressed(fin_v.at[pl.ds(p, 16)], v, mask=m)
                plsc.store_compressed(fidx_v.at[pl.ds(p, 16)], ix, mask=m)
                return p + plsc.all_reduce_population_count(m)[0]
            nhi = lax.fori_loop(0, nv_cand, hc_body, jnp.int32(0))

            def hm_body(i, p):
                v = map_v[pl.ds(i * 16, 16)]
                bits = plsc.bitcast(v, jnp.int32)
                m = bits > vkb
                plsc.store_compressed(fin_v.at[pl.ds(p, 16)], v, mask=m)
                plsc.store_compressed(fidx_v.at[pl.ds(p, 16)],
                                      laneiota + i * 16, mask=m)
                return p + plsc.all_reduce_population_count(m)[0]
            nhi = lax.fori_loop(0, nv_map, hm_body, nhi)

            def tc_body(i, p):
                v = cand_v[pl.ds(i * 16, 16)]
                ix = cidx_v[pl.ds(i * 16, 16)]
                bits = plsc.bitcast(v, jnp.int32)
                valid = (i * 16 + laneiota) < cnt
                m = (bits == vkb) & valid
                off = jnp.minimum(p, 100)
                plsc.store_compressed(tie_v.at[pl.ds(off, 16)], v, mask=m)
                plsc.store_compressed(tidx_v.at[pl.ds(off, 16)], ix, mask=m)
                return p + plsc.all_reduce_population_count(m)[0]
            nt = lax.fori_loop(0, nv_cand, tc_body, jnp.int32(0))

            def tm_body(i, p):
                v = map_v[pl.ds(i * 16, 16)]
                bits = plsc.bitcast(v, jnp.int32)
                m = bits == vkb
                off = jnp.minimum(p, 100)
                plsc.store_compressed(tie_v.at[pl.ds(off, 16)], v, mask=m)
                plsc.store_compressed(tidx_v.at[pl.ds(off, 16)],
                                      laneiota + i * 16, mask=m)
                return p + plsc.all_reduce_population_count(m)[0]
            lax.fori_loop(0, nv_map, tm_body, nt)

            for t in range(7):
                p = t * 16 + laneiota
                cur_v = fin_v[pl.ds(t * 16, 16)]
                cur_i = fidx_v[pl.ds(t * 16, 16)]
                src = jnp.maximum(p - nhi, 0)
                tv = plsc.load_gather(tie_v, [src])
                ti = plsc.load_gather(tidx_v, [src])
                in_hi = p < nhi
                in_tie = p < K
                nv = jnp.where(in_hi, cur_v,
                               jnp.where(in_tie, tv, jnp.float32(-1.0)))
                ni = jnp.where(in_hi, cur_i,
                               jnp.where(in_tie, ti, jnp.int32(0x7FFFFFFF)))
                fin_v[pl.ds(t * 16, 16)] = nv
                fidx_v[pl.ds(t * 16, 16)] = ni
                rank_v[pl.ds(t * 16, 16)] = jnp.zeros((16,), jnp.int32)

            def rbody(j, _):
                vj = fin_v[pl.ds(j, 16)][0]
                ij = fidx_v[pl.ds(j, 16)][0]
                for t in range(7):
                    fv = fin_v[pl.ds(t * 16, 16)]
                    fi = fidx_v[pl.ds(t * 16, 16)]
                    before = (vj > fv) | ((vj == fv) & (ij < fi))
                    acc = rank_v[pl.ds(t * 16, 16)]
                    rank_v[pl.ds(t * 16, 16)] = acc + before.astype(jnp.int32)
                return 0
            lax.fori_loop(0, K, rbody, 0)

            for t in range(7):
                r = rank_v[pl.ds(t * 16, 16)]
                m = r < K
                rc = jnp.minimum(r, jnp.int32(127))
                plsc.store_scatter(outv_v, [rc], fin_v[pl.ds(t * 16, 16)],
                                   mask=m)
                plsc.store_scatter(outi_v, [rc], fidx_v[pl.ds(t * 16, 16)],
                                   mask=m)

            pltpu.sync_copy(outv_v, outv_hbm.at[g])
            pltpu.sync_copy(outi_v, outi_hbm.at[g])

        def round_body(r, _):
            process(w + 32 * r)
            return 0
        nrounds = jnp.where(w < NMAPS - 128, 5, 4)
        lax.fori_loop(0, nrounds, round_body, 0)

    return topk_kernel(hm_sc, hp_sc)


KP = 104


def _decode_body(hx_ref, hy_ref, hs_ref, rx_ref, ry_ref,
                 l_ref, t_ref, r_ref, bo_ref, ox_ref, oy_ref):
    hx = hx_ref[0, 0]
    hy = hy_ref[0, 0]
    hs = hs_ref[0, 0]
    rx = rx_ref[0, 0]
    ry = ry_ref[0, 0]
    l = l_ref[0]
    t = t_ref[0]
    r = r_ref[0]
    bo = bo_ref[0]
    dx = rx - hx
    dy = ry - hy
    dist = jnp.sqrt(dx * dx + dy * dy)
    mind = jnp.min(dist, axis=1, keepdims=True)
    lane = lax.broadcasted_iota(jnp.int32, (KP, 128), 1)
    minidx = jnp.min(jnp.where(dist == mind, lane, jnp.int32(1 << 30)),
                     axis=1, keepdims=True)
    sel = lane == minidx
    zero = jnp.zeros((KP, 128), jnp.float32)
    hxsel = jnp.sum(jnp.where(sel, jnp.broadcast_to(hx, (KP, 128)), zero),
                    axis=1, keepdims=True)
    hysel = jnp.sum(jnp.where(sel, jnp.broadcast_to(hy, (KP, 128)), zero),
                    axis=1, keepdims=True)
    hssel = jnp.sum(jnp.where(sel, jnp.broadcast_to(hs, (KP, 128)), zero),
                    axis=1, keepdims=True)
    m2 = ((hxsel < l) | (hxsel > r) | (hysel < t) | (hysel > bo) |
          (hssel < jnp.float32(0.1)) |
          (mind > jnp.maximum(bo - t, r - l) * jnp.float32(0.3)))
    ox_ref[0, 0] = jnp.where(m2, rx, hxsel)
    oy_ref[0, 0] = jnp.where(m2, ry, hysel)


def _decode(hm_xs, hm_ys, hm_score, rx, ry, bboxes):
    bigf = jnp.float32(1e9)
    row = lambda a, pad: jnp.pad(a, ((0, 0), (0, 0), (0, 128 - K)),
                                 constant_values=pad)[:, :, None, :]
    col = lambda a: jnp.pad(a, ((0, 0), (0, 0), (0, KP - K)))[..., None]
    bcol = lambda a: jnp.pad(a, ((0, 0), (0, KP - K)))[..., None]
    hxr = row(hm_xs, bigf)
    hyr = row(hm_ys, bigf)
    hsr = row(hm_score, 0.0)
    rxc = col(rx)
    ryc = col(ry)
    l = bcol(bboxes[:, :, 0])
    t = bcol(bboxes[:, :, 1])
    r = bcol(bboxes[:, :, 2])
    bo = bcol(bboxes[:, :, 3])
    G = B * J
    rowspec = pl.BlockSpec((1, 1, 1, 128), lambda g: (g // J, g % J, 0, 0))
    colspec = pl.BlockSpec((1, 1, KP, 1), lambda g: (g // J, g % J, 0, 0))
    bspec = pl.BlockSpec((1, KP, 1), lambda g: (g // J, 0, 0))
    ox, oy = pl.pallas_call(
        _decode_body,
        grid=(G,),
        in_specs=[rowspec, rowspec, rowspec, colspec, colspec,
                  bspec, bspec, bspec, bspec],
        out_specs=[colspec, colspec],
        out_shape=[jax.ShapeDtypeStruct((B, J, KP, 1), jnp.float32)] * 2,
    )(hxr, hyr, hsr, rxc, ryc, l, t, r, bo)
    return ox[:, :, :K, 0], oy[:, :, :K, 0]


def _gather_feat(feat, ind):
    b, k = ind.shape
    c = feat.shape[2]
    idx = jnp.broadcast_to(ind[:, :, None], (b, k, c))
    return jnp.take_along_axis(feat, idx, axis=1)


def _transpose_gather(feat, ind):
    b, c, h, w = feat.shape
    feat = jnp.transpose(feat, (0, 2, 3, 1)).reshape(b, h * w, c)
    return _gather_feat(feat, ind)


def kernel(hm, wh, hps, reg, hm_hp, hp_offset):
    hm_s, hm_scores = _sig_nms(hm)
    hm_hp_s, hp_scores = _sig_nms(hm_hp)

    vals, idxs = _sc_topk(hm_scores, hp_scores)
    b = B
    scores = vals[:B, :K]
    inds = idxs[:B, :K]
    hm_score = vals[B:, :K].reshape(b, J, K)
    hm_inds = idxs[B:, :K].reshape(b, J, K)

    ys = (inds // W).astype(jnp.float32)
    xs = (inds % W).astype(jnp.float32)
    clses2 = jnp.zeros((b, K, 1), jnp.float32)

    kps = _transpose_gather(hps, inds)
    kps = kps.at[..., 0::2].add(xs[:, :, None])
    kps = kps.at[..., 1::2].add(ys[:, :, None])
    regg = _transpose_gather(reg, inds)
    xs2 = xs[:, :, None] + regg[:, :, 0:1]
    ys2 = ys[:, :, None] + regg[:, :, 1:2]
    whg = _transpose_gather(wh, inds)
    scores2 = scores[:, :, None]
    bboxes = jnp.concatenate([
        xs2 - whg[..., 0:1] / 2, ys2 - whg[..., 1:2] / 2,
        xs2 + whg[..., 0:1] / 2, ys2 + whg[..., 1:2] / 2], axis=2)
    thresh = 0.1
    kps_t = jnp.transpose(kps.reshape(b, K, J, 2), (0, 2, 1, 3))
    rx = kps_t[..., 0]
    ry = kps_t[..., 1]
    hm_ys = (hm_inds // W).astype(jnp.float32)
    hm_xs = (hm_inds % W).astype(jnp.float32)
    hp_off = _transpose_gather(hp_offset, hm_inds.reshape(b, -1)).reshape(b, J, K, 2)
    hm_xs = hm_xs + hp_off[..., 0]
    hm_ys = hm_ys + hp_off[..., 1]
    mask = (hm_score > thresh).astype(jnp.float32)
    hm_score = (1 - mask) * -1 + mask * hm_score
    hm_ys = (1 - mask) * -10000 + mask * hm_ys
    hm_xs = (1 - mask) * -10000 + mask * hm_xs
    ox, oy = _decode(hm_xs, hm_ys, hm_score, rx, ry, bboxes)
    kps_f = jnp.stack([ox, oy], axis=-1)
    kps_f = jnp.transpose(kps_f, (0, 2, 1, 3)).reshape(b, K, J * 2)
    det = jnp.concatenate([bboxes, scores2, kps_f, clses2], axis=2)
    return (hm_s, wh, hps, reg, hm_hp_s, hp_offset, det)
